# baseline (speedup 1.0000x reference)
import functools
import jax
import jax.numpy as jnp
from jax import lax
from jax.experimental import pallas as pl
from jax.experimental.pallas import tpu as pltpu
from jax.experimental.pallas import tpu_sc as plsc

D_MODEL = 1024
D_FF = 4096
NUM_EXPERTS = 8
TOP_K = 2
N_TOKENS = 2048
NK = N_TOKENS * TOP_K
EPS = 1e-8
CLAMP_MIN, CLAMP_MAX = -10000.0, 10000.0

TM = 256
TF = 1024
NF = D_FF // TF
PAD_ROWS = NK + NUM_EXPERTS * TM
NT = PAD_ROWS // TM
RT = 128
E_PAD = 128

NWORKERS = 32
CH = NK // NWORKERS
TOK_CH = N_TOKENS // NWORKERS
BS = 16
NB = CH // BS


def _router_body(x_ref, wr_ref, i1_ref, i2_ref, p1_ref, p2_ref):
    xb = x_ref[...]
    wr = wr_ref[...]
    logits = lax.dot_general(xb, wr, (((1,), (1,)), ((), ())),
                             preferred_element_type=jnp.float32)
    col = lax.broadcasted_iota(jnp.int32, logits.shape, 1)
    valid = col < NUM_EXPERTS
    lg = jnp.clip(logits, CLAMP_MIN, CLAMP_MAX)
    lg = jnp.where(valid, lg, -1e30)
    m = jnp.max(lg, axis=1, keepdims=True)
    ex = jnp.where(valid, jnp.exp(lg - m), 0.0)
    s = jnp.sum(ex, axis=1, keepdims=True)
    probs = ex / (s + EPS)
    probs = jnp.where(valid, jnp.clip(probs, EPS, 1.0), -1.0)
    m1 = jnp.max(probs, axis=1, keepdims=True)
    i1 = jnp.min(jnp.where(probs == m1, col, E_PAD), axis=1, keepdims=True)
    probs2 = jnp.where(col == i1, -2.0, probs)
    m2 = jnp.max(probs2, axis=1, keepdims=True)
    i2 = jnp.min(jnp.where(probs2 == m2, col, E_PAD), axis=1, keepdims=True)
    denom = m1 + m2 + EPS
    i1_ref[...] = i1
    i2_ref[...] = i2
    p1_ref[...] = m1 / denom
    p2_ref[...] = m2 / denom


def _run_router(x, W_router):
    wr_pad = jnp.zeros((E_PAD, D_MODEL), jnp.float32).at[:NUM_EXPERTS].set(W_router)
    grid = (N_TOKENS // RT,)
    out_shapes = [
        jax.ShapeDtypeStruct((N_TOKENS, 1), jnp.int32),
        jax.ShapeDtypeStruct((N_TOKENS, 1), jnp.int32),
        jax.ShapeDtypeStruct((N_TOKENS, 1), jnp.float32),
        jax.ShapeDtypeStruct((N_TOKENS, 1), jnp.float32),
    ]
    o_spec = pl.BlockSpec((RT, 1), lambda t: (t, 0))
    return pl.pallas_call(
        _router_body,
        grid=grid,
        in_specs=[
            pl.BlockSpec((RT, D_MODEL), lambda t: (t, 0)),
            pl.BlockSpec((E_PAD, D_MODEL), lambda t: (0, 0)),
        ],
        out_specs=[o_spec, o_spec, o_spec, o_spec],
        out_shape=out_shapes,
    )(x, wr_pad)


def _dispatch_body(eid_hbm, pr_hbm, x_hbm,
                   dst_hbm, xd_hbm, sinfo_hbm,
                   eid_v, pr_v, dst_v, idxg, idxs, rowbuf, sinfo_v, sem):
    wid = lax.axis_index("s") * 2 + lax.axis_index("c")
    base = wid * CH
    iota = lax.iota(jnp.int32, 16)

    pltpu.sync_copy(eid_hbm, eid_v)
    pltpu.sync_copy(pr_hbm.at[pl.ds(base, CH)], pr_v)

    zero = jnp.int32(0)

    def scan_body(v, carry):
        tots, priors = carry
        vec = eid_v[pl.ds(v * 16, 16)]
        inb = ((v * 16 + iota) < base).astype(jnp.int32)
        new_t = []
        new_p = []
        for e in range(NUM_EXPERTS):
            eq = (vec == e).astype(jnp.int32)
            new_t.append(tots[e] + jnp.sum(eq))
            new_p.append(priors[e] + jnp.sum(eq * inb))
        return tuple(new_t), tuple(new_p)

    tots, priors = lax.fori_loop(
        0, NK // 16, scan_body,
        (tuple([zero] * NUM_EXPERTS), tuple([zero] * NUM_EXPERTS)))

    pad_off = [zero]
    for e in range(NUM_EXPERTS):
        padc = (tots[e] + (TM - 1)) & ~jnp.int32(TM - 1)
        pad_off.append(pad_off[e] + padc)

    run = [pad_off[e] + priors[e] for e in range(NUM_EXPERTS)]
    for bv in range(NB):
        vec = eid_v[pl.ds(base + bv * 16, 16)]
        dstv = jnp.zeros((16,), jnp.int32)
        for e in range(NUM_EXPERTS):
            eq = vec == e
            c = plsc.cumsum(eq.astype(jnp.int32))
            dstv = jnp.where(eq, run[e] + c - 1, dstv)
            run[e] = run[e] + jnp.sum(eq.astype(jnp.int32))
        dst_v[pl.ds(bv * 16, 16)] = dstv
    pltpu.sync_copy(dst_v, dst_hbm.at[pl.ds(base, CH)])

    @pl.when(wid == 0)
    def _():
        for half in range(2):
            starts = (iota + half * 16) * TM
            te = jnp.zeros((16,), jnp.int32)
            for e in range(NUM_EXPERTS):
                te = te + (starts >= pad_off[e + 1]).astype(jnp.int32)
            te = jnp.minimum(te, NUM_EXPERTS - 1)
            tv = (starts < pad_off[NUM_EXPERTS]).astype(jnp.int32)
            sinfo_v[0, pl.ds(half * 16, 16)] = te
            sinfo_v[1, pl.ds(half * 16, 16)] = tv
        pltpu.sync_copy(sinfo_v, sinfo_hbm)

    for b in range(NB):
        idxg[b, :] = lax.shift_right_logical(base + b * 16 + iota, 1)
        idxs[b, :] = dst_v[pl.ds(b * 16, 16)]
        pltpu.async_copy(x_hbm.at[idxg.at[b]], rowbuf, sem).wait()

        def scale_row(r, carry):
            pidx = jnp.zeros((16,), jnp.int32) + (b * 16 + r)
            psp = plsc.load_gather(pr_v, [pidx])
            for j in range(D_MODEL // 16):
                rowbuf[r, pl.ds(j * 16, 16)] = \
                    rowbuf[r, pl.ds(j * 16, 16)] * psp
            return carry

        lax.fori_loop(0, BS, scale_row, 0)
        pltpu.async_copy(rowbuf, xd_hbm.at[idxs.at[b]], sem).wait()


def _run_dispatch(eid, pr, x):
    mesh = plsc.VectorSubcoreMesh(core_axis_name="c", subcore_axis_name="s",
                                   num_cores=2, num_subcores=16)
    f = pl.kernel(
        _dispatch_body,
        out_type=[
            jax.ShapeDtypeStruct((NK,), jnp.int32),
            jax.ShapeDtypeStruct((PAD_ROWS, D_MODEL), jnp.float32),
            jax.ShapeDtypeStruct((2, 32), jnp.int32),
        ],
        mesh=mesh,
        scratch_types=[
            pltpu.VMEM((NK,), jnp.int32),
            pltpu.VMEM((CH,), jnp.float32),
            pltpu.VMEM((CH,), jnp.int32),
            pltpu.VMEM((NB, 16), jnp.int32),
            pltpu.VMEM((NB, 16), jnp.int32),
            pltpu.VMEM((BS, D_MODEL), jnp.float32),
            pltpu.VMEM((2, 32), jnp.int32),
            pltpu.SemaphoreType.DMA,
        ],
        compiler_params=pltpu.CompilerParams(needs_layout_passes=False),
    )
    return f(eid, pr, x)


def _ffn_body(sref, x_ref, wg_ref, wu_ref, wd_ref, y_ref, acc_ref):
    f = pl.program_id(0)
    t = pl.program_id(1)
    valid = sref[1, t] == 1

    @pl.when(valid)
    def _():
        xb = x_ref[...].astype(jnp.bfloat16)
        g = lax.dot_general(xb, wg_ref[0], (((1,), (1,)), ((), ())),
                            preferred_element_type=jnp.float32)
        u = lax.dot_general(xb, wu_ref[0], (((1,), (1,)), ((), ())),
                            preferred_element_type=jnp.float32)
        h = ((g * jax.nn.sigmoid(g)) * u).astype(jnp.bfloat16)
        yp = lax.dot_general(h, wd_ref[0], (((1,), (1,)), ((), ())),
                             preferred_element_type=jnp.float32)

        @pl.when(f == 0)
        def _():
            acc_ref[t] = yp

        @pl.when(jnp.logical_and(f > 0, f < NF - 1))
        def _():
            acc_ref[t] = acc_ref[t] + yp

        @pl.when(f == NF - 1)
        def _():
            y_ref[...] = acc_ref[t] + yp


def _run_ffn(sinfo, X_disp, W_gu, W_d):
    grid = (NF, NT)
    grid_spec = pltpu.PrefetchScalarGridSpec(
        num_scalar_prefetch=1,
        grid=grid,
        in_specs=[
            pl.BlockSpec((TM, D_MODEL), lambda f, t, s: (t, 0)),
            pl.BlockSpec((1, TF, D_MODEL), lambda f, t, s: (s[0, t], f, 0)),
            pl.BlockSpec((1, TF, D_MODEL), lambda f, t, s: (s[0, t], NF + f, 0)),
            pl.BlockSpec((1, D_MODEL, TF), lambda f, t, s: (s[0, t], 0, f)),
        ],
        out_specs=pl.BlockSpec((TM, D_MODEL), lambda f, t, s: (t, 0)),
        scratch_shapes=[pltpu.VMEM((NT, TM, D_MODEL), jnp.float32)],
    )
    return pl.pallas_call(
        _ffn_body,
        grid_spec=grid_spec,
        out_shape=jax.ShapeDtypeStruct((PAD_ROWS, D_MODEL), jnp.float32),
        compiler_params=pltpu.CompilerParams(
            dimension_semantics=("arbitrary", "arbitrary"),
        ),
    )(sinfo, X_disp, W_gu, W_gu, W_d)


def _combine_body(y_hbm, dst_hbm, out_hbm, dst_v, idxr, ybuf, obuf, sem):
    wid = lax.axis_index("s") * 2 + lax.axis_index("c")
    base = wid * CH
    tok_base = wid * TOK_CH

    pltpu.sync_copy(dst_hbm.at[pl.ds(base, CH)], dst_v)
    for b in range(NB):
        idxr[b, :] = dst_v[pl.ds(b * 16, 16)]
        pltpu.async_copy(y_hbm.at[idxr.at[b]], ybuf, sem).wait()

        def add_row(r, carry):
            for j in range(D_MODEL // 16):
                sl = pl.ds(j * 16, 16)
                obuf[r, sl] = ybuf[2 * r, sl] + ybuf[2 * r + 1, sl]
            return carry

        lax.fori_loop(0, BS // TOP_K, add_row, 0)
        pltpu.sync_copy(
            obuf, out_hbm.at[pl.ds(tok_base + b * (BS // TOP_K), BS // TOP_K)])


def _run_combine(Y_disp, dst):
    mesh = plsc.VectorSubcoreMesh(core_axis_name="c", subcore_axis_name="s",
                                   num_cores=2, num_subcores=16)
    f = pl.kernel(
        _combine_body,
        out_type=jax.ShapeDtypeStruct((N_TOKENS, D_MODEL), jnp.float32),
        mesh=mesh,
        scratch_types=[
            pltpu.VMEM((CH,), jnp.int32),
            pltpu.VMEM((NB, 16), jnp.int32),
            pltpu.VMEM((BS, D_MODEL), jnp.float32),
            pltpu.VMEM((BS // TOP_K, D_MODEL), jnp.float32),
            pltpu.SemaphoreType.DMA,
        ],
        compiler_params=pltpu.CompilerParams(needs_layout_passes=False),
    )
    return f(Y_disp, dst)


def kernel(x, W_router, W_gu, W_d):
    i1, i2, p1, p2 = _run_router(x, W_router)
    eid = jnp.concatenate([i1, i2], axis=1).reshape(-1)
    pr = jnp.concatenate([p1, p2], axis=1).reshape(-1)

    dst, X_disp, sinfo_p = _run_dispatch(eid, pr, x)
    sinfo = sinfo_p[:, :NT]

    Y_disp = _run_ffn(sinfo, X_disp,
                      W_gu.astype(jnp.bfloat16), W_d.astype(jnp.bfloat16))
    return _run_combine(Y_disp, dst)


# speedup vs baseline: 1.2868x; 1.2868x over previous
import functools
import jax
import jax.numpy as jnp
from jax import lax
from jax.experimental import pallas as pl
from jax.experimental.pallas import tpu as pltpu
from jax.experimental.pallas import tpu_sc as plsc

D_MODEL = 1024
D_FF = 4096
NUM_EXPERTS = 8
TOP_K = 2
N_TOKENS = 2048
NK = N_TOKENS * TOP_K
EPS = 1e-8
CLAMP_MIN, CLAMP_MAX = -10000.0, 10000.0

TM = 256
TF = 1024
NF = D_FF // TF
PAD_ROWS = NK + NUM_EXPERTS * TM
NT = PAD_ROWS // TM
RT = 128
E_PAD = 128

NWORKERS = 32
CH = NK // NWORKERS
TOK_CH = N_TOKENS // NWORKERS
BS = 16
NB = CH // BS


def _router_body(x_ref, wr_ref, i1_ref, i2_ref, p1_ref, p2_ref):
    xb = x_ref[...]
    wr = wr_ref[...]
    logits = lax.dot_general(xb, wr, (((1,), (1,)), ((), ())),
                             preferred_element_type=jnp.float32)
    col = lax.broadcasted_iota(jnp.int32, logits.shape, 1)
    valid = col < NUM_EXPERTS
    lg = jnp.clip(logits, CLAMP_MIN, CLAMP_MAX)
    lg = jnp.where(valid, lg, -1e30)
    m = jnp.max(lg, axis=1, keepdims=True)
    ex = jnp.where(valid, jnp.exp(lg - m), 0.0)
    s = jnp.sum(ex, axis=1, keepdims=True)
    probs = ex / (s + EPS)
    probs = jnp.where(valid, jnp.clip(probs, EPS, 1.0), -1.0)
    m1 = jnp.max(probs, axis=1, keepdims=True)
    i1 = jnp.min(jnp.where(probs == m1, col, E_PAD), axis=1, keepdims=True)
    probs2 = jnp.where(col == i1, -2.0, probs)
    m2 = jnp.max(probs2, axis=1, keepdims=True)
    i2 = jnp.min(jnp.where(probs2 == m2, col, E_PAD), axis=1, keepdims=True)
    denom = m1 + m2 + EPS
    i1_ref[...] = i1
    i2_ref[...] = i2
    p1_ref[...] = m1 / denom
    p2_ref[...] = m2 / denom


def _run_router(x, W_router):
    wr_pad = jnp.zeros((E_PAD, D_MODEL), jnp.float32).at[:NUM_EXPERTS].set(W_router)
    grid = (N_TOKENS // RT,)
    out_shapes = [
        jax.ShapeDtypeStruct((N_TOKENS, 1), jnp.int32),
        jax.ShapeDtypeStruct((N_TOKENS, 1), jnp.int32),
        jax.ShapeDtypeStruct((N_TOKENS, 1), jnp.float32),
        jax.ShapeDtypeStruct((N_TOKENS, 1), jnp.float32),
    ]
    o_spec = pl.BlockSpec((RT, 1), lambda t: (t, 0))
    return pl.pallas_call(
        _router_body,
        grid=grid,
        in_specs=[
            pl.BlockSpec((RT, D_MODEL), lambda t: (t, 0)),
            pl.BlockSpec((E_PAD, D_MODEL), lambda t: (0, 0)),
        ],
        out_specs=[o_spec, o_spec, o_spec, o_spec],
        out_shape=out_shapes,
    )(x, wr_pad)


def _dispatch_body(eid_hbm, pr_hbm, x_hbm,
                   dst_hbm, xd_hbm, sinfo_hbm,
                   eid_v, pr_v, dst_v, idxg, idxs, rowbuf, sinfo_v,
                   gsem, ssem):
    wid = lax.axis_index("s") * 2 + lax.axis_index("c")
    base = wid * CH
    iota = lax.iota(jnp.int32, 16)

    pltpu.sync_copy(eid_hbm, eid_v)
    pltpu.sync_copy(pr_hbm.at[pl.ds(base, CH)], pr_v)

    zero = jnp.int32(0)

    def scan_body(v, carry):
        tots, priors = carry
        vec = eid_v[pl.ds(v * 16, 16)]
        inb = ((v * 16 + iota) < base).astype(jnp.int32)
        new_t = []
        new_p = []
        for e in range(NUM_EXPERTS):
            eq = (vec == e).astype(jnp.int32)
            new_t.append(tots[e] + jnp.sum(eq))
            new_p.append(priors[e] + jnp.sum(eq * inb))
        return tuple(new_t), tuple(new_p)

    tots, priors = lax.fori_loop(
        0, NK // 16, scan_body,
        (tuple([zero] * NUM_EXPERTS), tuple([zero] * NUM_EXPERTS)))

    pad_off = [zero]
    for e in range(NUM_EXPERTS):
        padc = (tots[e] + (TM - 1)) & ~jnp.int32(TM - 1)
        pad_off.append(pad_off[e] + padc)

    run = [pad_off[e] + priors[e] for e in range(NUM_EXPERTS)]
    for bv in range(NB):
        vec = eid_v[pl.ds(base + bv * 16, 16)]
        dstv = jnp.zeros((16,), jnp.int32)
        for e in range(NUM_EXPERTS):
            eq = vec == e
            c = plsc.cumsum(eq.astype(jnp.int32))
            dstv = jnp.where(eq, run[e] + c - 1, dstv)
            run[e] = run[e] + jnp.sum(eq.astype(jnp.int32))
        dst_v[pl.ds(bv * 16, 16)] = dstv
    pltpu.sync_copy(dst_v, dst_hbm.at[pl.ds(base, CH)])

    @pl.when(wid == 0)
    def _():
        for half in range(2):
            starts = (iota + half * 16) * TM
            te = jnp.zeros((16,), jnp.int32)
            for e in range(NUM_EXPERTS):
                te = te + (starts >= pad_off[e + 1]).astype(jnp.int32)
            te = jnp.minimum(te, NUM_EXPERTS - 1)
            tv = (starts < pad_off[NUM_EXPERTS]).astype(jnp.int32)
            sinfo_v[0, pl.ds(half * 16, 16)] = te
            sinfo_v[1, pl.ds(half * 16, 16)] = tv
        pltpu.sync_copy(sinfo_v, sinfo_hbm)

    for b in range(NB):
        idxg[b, :] = lax.shift_right_logical(base + b * 16 + iota, 1)
        idxs[b, :] = dst_v[pl.ds(b * 16, 16)]

    gh = [None] * NB
    sh = [None] * NB
    gh[0] = pltpu.async_copy(x_hbm.at[idxg.at[0]], rowbuf.at[0], gsem)
    for b in range(NB):
        cur = b % 2
        gh[b].wait()
        if b + 1 < NB:
            if b >= 1:
                sh[b - 1].wait()
            gh[b + 1] = pltpu.async_copy(
                x_hbm.at[idxg.at[b + 1]], rowbuf.at[(b + 1) % 2], gsem)

        def scale_row(r, carry):
            pidx = jnp.zeros((16,), jnp.int32) + (b * 16 + r)
            psp = plsc.load_gather(pr_v, [pidx])
            for j in range(D_MODEL // 16):
                rowbuf[cur, r, pl.ds(j * 16, 16)] = \
                    rowbuf[cur, r, pl.ds(j * 16, 16)] * psp
            return carry

        lax.fori_loop(0, BS, scale_row, 0)
        sh[b] = pltpu.async_copy(rowbuf.at[cur], xd_hbm.at[idxs.at[b]], ssem)
    sh[NB - 2].wait()
    sh[NB - 1].wait()


def _run_dispatch(eid, pr, x):
    mesh = plsc.VectorSubcoreMesh(core_axis_name="c", subcore_axis_name="s",
                                   num_cores=2, num_subcores=16)
    f = pl.kernel(
        _dispatch_body,
        out_type=[
            jax.ShapeDtypeStruct((NK,), jnp.int32),
            jax.ShapeDtypeStruct((PAD_ROWS, D_MODEL), jnp.float32),
            jax.ShapeDtypeStruct((2, 32), jnp.int32),
        ],
        mesh=mesh,
        scratch_types=[
            pltpu.VMEM((NK,), jnp.int32),
            pltpu.VMEM((CH,), jnp.float32),
            pltpu.VMEM((CH,), jnp.int32),
            pltpu.VMEM((NB, 16), jnp.int32),
            pltpu.VMEM((NB, 16), jnp.int32),
            pltpu.VMEM((2, BS, D_MODEL), jnp.float32),
            pltpu.VMEM((2, 32), jnp.int32),
            pltpu.SemaphoreType.DMA,
            pltpu.SemaphoreType.DMA,
        ],
        compiler_params=pltpu.CompilerParams(needs_layout_passes=False),
    )
    return f(eid, pr, x)


def _ffn_body(sref, x_ref, wg_ref, wu_ref, wd_ref, y_ref, acc_ref):
    f = pl.program_id(0)
    t = pl.program_id(1)
    valid = sref[1, t] == 1

    @pl.when(valid)
    def _():
        xb = x_ref[...]
        g = lax.dot_general(xb, wg_ref[0], (((1,), (1,)), ((), ())),
                            preferred_element_type=jnp.float32)
        u = lax.dot_general(xb, wu_ref[0], (((1,), (1,)), ((), ())),
                            preferred_element_type=jnp.float32)
        h = (g * jax.nn.sigmoid(g)) * u
        yp = lax.dot_general(h, wd_ref[0], (((1,), (1,)), ((), ())),
                             preferred_element_type=jnp.float32)

        @pl.when(f == 0)
        def _():
            acc_ref[t] = yp

        @pl.when(jnp.logical_and(f > 0, f < NF - 1))
        def _():
            acc_ref[t] = acc_ref[t] + yp

        @pl.when(f == NF - 1)
        def _():
            y_ref[...] = acc_ref[t] + yp


def _run_ffn(sinfo, X_disp, W_gu, W_d):
    grid = (NF, NT)
    grid_spec = pltpu.PrefetchScalarGridSpec(
        num_scalar_prefetch=1,
        grid=grid,
        in_specs=[
            pl.BlockSpec((TM, D_MODEL), lambda f, t, s: (t, 0)),
            pl.BlockSpec((1, TF, D_MODEL), lambda f, t, s: (s[0, t], f, 0)),
            pl.BlockSpec((1, TF, D_MODEL), lambda f, t, s: (s[0, t], NF + f, 0)),
            pl.BlockSpec((1, D_MODEL, TF), lambda f, t, s: (s[0, t], 0, f)),
        ],
        out_specs=pl.BlockSpec((TM, D_MODEL), lambda f, t, s: (t, 0)),
        scratch_shapes=[pltpu.VMEM((NT, TM, D_MODEL), jnp.float32)],
    )
    return pl.pallas_call(
        _ffn_body,
        grid_spec=grid_spec,
        out_shape=jax.ShapeDtypeStruct((PAD_ROWS, D_MODEL), jnp.float32),
        compiler_params=pltpu.CompilerParams(
            dimension_semantics=("arbitrary", "arbitrary"),
        ),
    )(sinfo, X_disp, W_gu, W_gu, W_d)


def _combine_body(y_hbm, dst_hbm, out_hbm, dst_v, idxr, ybuf, obuf,
                  gsem, ssem):
    wid = lax.axis_index("s") * 2 + lax.axis_index("c")
    base = wid * CH
    tok_base = wid * TOK_CH

    pltpu.sync_copy(dst_hbm.at[pl.ds(base, CH)], dst_v)
    for b in range(NB):
        idxr[b, :] = dst_v[pl.ds(b * 16, 16)]

    gh = [None] * NB
    sh = [None] * NB
    gh[0] = pltpu.async_copy(y_hbm.at[idxr.at[0]], ybuf.at[0], gsem)
    for b in range(NB):
        cur = b % 2
        gh[b].wait()
        if b + 1 < NB:
            if b >= 1:
                sh[b - 1].wait()
            gh[b + 1] = pltpu.async_copy(
                y_hbm.at[idxr.at[b + 1]], ybuf.at[(b + 1) % 2], gsem)

        def add_row(r, carry):
            for j in range(D_MODEL // 16):
                sl = pl.ds(j * 16, 16)
                obuf[cur, r, sl] = ybuf[cur, 2 * r, sl] + ybuf[cur, 2 * r + 1, sl]
            return carry

        lax.fori_loop(0, BS // TOP_K, add_row, 0)
        sh[b] = pltpu.async_copy(
            obuf.at[cur],
            out_hbm.at[pl.ds(tok_base + b * (BS // TOP_K), BS // TOP_K)],
            ssem)
    sh[NB - 2].wait()
    sh[NB - 1].wait()


def _run_combine(Y_disp, dst):
    mesh = plsc.VectorSubcoreMesh(core_axis_name="c", subcore_axis_name="s",
                                   num_cores=2, num_subcores=16)
    f = pl.kernel(
        _combine_body,
        out_type=jax.ShapeDtypeStruct((N_TOKENS, D_MODEL), jnp.float32),
        mesh=mesh,
        scratch_types=[
            pltpu.VMEM((CH,), jnp.int32),
            pltpu.VMEM((NB, 16), jnp.int32),
            pltpu.VMEM((2, BS, D_MODEL), jnp.float32),
            pltpu.VMEM((2, BS // TOP_K, D_MODEL), jnp.float32),
            pltpu.SemaphoreType.DMA,
            pltpu.SemaphoreType.DMA,
        ],
        compiler_params=pltpu.CompilerParams(needs_layout_passes=False),
    )
    return f(Y_disp, dst)


def kernel(x, W_router, W_gu, W_d):
    i1, i2, p1, p2 = _run_router(x, W_router)
    eid = jnp.concatenate([i1, i2], axis=1).reshape(-1)
    pr = jnp.concatenate([p1, p2], axis=1).reshape(-1)

    dst, X_disp, sinfo_p = _run_dispatch(eid, pr, x)
    sinfo = sinfo_p[:, :NT]

    Y_disp = _run_ffn(sinfo, X_disp, W_gu, W_d)
    return _run_combine(Y_disp, dst)


# speedup vs baseline: 1.3082x; 1.0166x over previous
import functools
import jax
import jax.numpy as jnp
from jax import lax
from jax.experimental import pallas as pl
from jax.experimental.pallas import tpu as pltpu
from jax.experimental.pallas import tpu_sc as plsc

D_MODEL = 1024
D_FF = 4096
NUM_EXPERTS = 8
TOP_K = 2
N_TOKENS = 2048
NK = N_TOKENS * TOP_K
EPS = 1e-8
CLAMP_MIN, CLAMP_MAX = -10000.0, 10000.0

TM = 256
TF = 1024
NF = D_FF // TF
PAD_ROWS = NK + NUM_EXPERTS * TM
NT = PAD_ROWS // TM
RT = 128
E_PAD = 128

NWORKERS = 32
CH = NK // NWORKERS
TOK_CH = N_TOKENS // NWORKERS
BS = 16
NB = CH // BS


def _router_body(x_ref, wr_ref, ei_ref, pp_ref):
    xb = x_ref[...]
    wr = wr_ref[...]
    logits = lax.dot_general(xb, wr, (((1,), (1,)), ((), ())),
                             preferred_element_type=jnp.float32)
    col = lax.broadcasted_iota(jnp.int32, logits.shape, 1)
    valid = col < NUM_EXPERTS
    lg = jnp.clip(logits, CLAMP_MIN, CLAMP_MAX)
    lg = jnp.where(valid, lg, -1e30)
    m = jnp.max(lg, axis=1, keepdims=True)
    ex = jnp.where(valid, jnp.exp(lg - m), 0.0)
    s = jnp.sum(ex, axis=1, keepdims=True)
    probs = ex / (s + EPS)
    probs = jnp.where(valid, jnp.clip(probs, EPS, 1.0), -1.0)
    m1 = jnp.max(probs, axis=1, keepdims=True)
    i1 = jnp.min(jnp.where(probs == m1, col, E_PAD), axis=1, keepdims=True)
    probs2 = jnp.where(col == i1, -2.0, probs)
    m2 = jnp.max(probs2, axis=1, keepdims=True)
    i2 = jnp.min(jnp.where(probs2 == m2, col, E_PAD), axis=1, keepdims=True)
    denom = m1 + m2 + EPS
    ei_ref[:, 0:1] = i1
    ei_ref[:, 1:2] = i2
    pp_ref[:, 0:1] = m1 / denom
    pp_ref[:, 1:2] = m2 / denom


def _run_router(x, W_router):
    wr_pad = jnp.zeros((E_PAD, D_MODEL), jnp.float32).at[:NUM_EXPERTS].set(W_router)
    grid = (N_TOKENS // RT,)
    out_shapes = [
        jax.ShapeDtypeStruct((N_TOKENS, 2), jnp.int32),
        jax.ShapeDtypeStruct((N_TOKENS, 2), jnp.float32),
    ]
    o_spec = pl.BlockSpec((RT, 2), lambda t: (t, 0))
    return pl.pallas_call(
        _router_body,
        grid=grid,
        in_specs=[
            pl.BlockSpec((RT, D_MODEL), lambda t: (t, 0)),
            pl.BlockSpec((E_PAD, D_MODEL), lambda t: (0, 0)),
        ],
        out_specs=[o_spec, o_spec],
        out_shape=out_shapes,
    )(x, wr_pad)


def _dispatch_body(eid_hbm, pr_hbm, x_hbm,
                   dst_hbm, xd_hbm, sinfo_hbm,
                   eid_v, pr_v, dst_v, idxg, idxs, rowbuf, sinfo_v,
                   gsem, ssem):
    wid = lax.axis_index("s") * 2 + lax.axis_index("c")
    base = wid * CH
    iota = lax.iota(jnp.int32, 16)

    pltpu.sync_copy(eid_hbm, eid_v)
    pltpu.sync_copy(pr_hbm.at[pl.ds(base, CH)], pr_v)

    zero = jnp.int32(0)

    def scan_body(v, carry):
        tots, priors = carry
        vec = eid_v[pl.ds(v * 16, 16)]
        inb = ((v * 16 + iota) < base).astype(jnp.int32)
        new_t = []
        new_p = []
        for e in range(NUM_EXPERTS):
            eq = (vec == e).astype(jnp.int32)
            new_t.append(tots[e] + jnp.sum(eq))
            new_p.append(priors[e] + jnp.sum(eq * inb))
        return tuple(new_t), tuple(new_p)

    tots, priors = lax.fori_loop(
        0, NK // 16, scan_body,
        (tuple([zero] * NUM_EXPERTS), tuple([zero] * NUM_EXPERTS)))

    pad_off = [zero]
    for e in range(NUM_EXPERTS):
        padc = (tots[e] + (TM - 1)) & ~jnp.int32(TM - 1)
        pad_off.append(pad_off[e] + padc)

    run = [pad_off[e] + priors[e] for e in range(NUM_EXPERTS)]
    for bv in range(NB):
        vec = eid_v[pl.ds(base + bv * 16, 16)]
        dstv = jnp.zeros((16,), jnp.int32)
        for e in range(NUM_EXPERTS):
            eq = vec == e
            c = plsc.cumsum(eq.astype(jnp.int32))
            dstv = jnp.where(eq, run[e] + c - 1, dstv)
            run[e] = run[e] + jnp.sum(eq.astype(jnp.int32))
        dst_v[pl.ds(bv * 16, 16)] = dstv
    pltpu.sync_copy(dst_v, dst_hbm.at[pl.ds(base, CH)])

    @pl.when(wid == 0)
    def _():
        for half in range(2):
            starts = (iota + half * 16) * TM
            te = jnp.zeros((16,), jnp.int32)
            for e in range(NUM_EXPERTS):
                te = te + (starts >= pad_off[e + 1]).astype(jnp.int32)
            te = jnp.minimum(te, NUM_EXPERTS - 1)
            tv = (starts < pad_off[NUM_EXPERTS]).astype(jnp.int32)
            sinfo_v[0, pl.ds(half * 16, 16)] = te
            sinfo_v[1, pl.ds(half * 16, 16)] = tv
        pltpu.sync_copy(sinfo_v, sinfo_hbm)

    for b in range(NB):
        idxg[b, :] = lax.shift_right_logical(base + b * 16 + iota, 1)
        idxs[b, :] = dst_v[pl.ds(b * 16, 16)]

    gh = [None] * NB
    sh = [None] * NB
    gh[0] = pltpu.async_copy(x_hbm.at[idxg.at[0]], rowbuf.at[0], gsem)
    for b in range(NB):
        cur = b % 2
        gh[b].wait()
        if b + 1 < NB:
            if b >= 1:
                sh[b - 1].wait()
            gh[b + 1] = pltpu.async_copy(
                x_hbm.at[idxg.at[b + 1]], rowbuf.at[(b + 1) % 2], gsem)

        def scale_row(r, carry):
            pidx = jnp.zeros((16,), jnp.int32) + (b * 16 + r)
            psp = plsc.load_gather(pr_v, [pidx])
            for j in range(D_MODEL // 16):
                rowbuf[cur, r, pl.ds(j * 16, 16)] = \
                    rowbuf[cur, r, pl.ds(j * 16, 16)] * psp
            return carry

        lax.fori_loop(0, BS, scale_row, 0)
        sh[b] = pltpu.async_copy(rowbuf.at[cur], xd_hbm.at[idxs.at[b]], ssem)
    sh[NB - 2].wait()
    sh[NB - 1].wait()


def _run_dispatch(eid, pr, x):
    mesh = plsc.VectorSubcoreMesh(core_axis_name="c", subcore_axis_name="s",
                                   num_cores=2, num_subcores=16)
    f = pl.kernel(
        _dispatch_body,
        out_type=[
            jax.ShapeDtypeStruct((NK,), jnp.int32),
            jax.ShapeDtypeStruct((PAD_ROWS, D_MODEL), jnp.float32),
            jax.ShapeDtypeStruct((2, 32), jnp.int32),
        ],
        mesh=mesh,
        scratch_types=[
            pltpu.VMEM((NK,), jnp.int32),
            pltpu.VMEM((CH,), jnp.float32),
            pltpu.VMEM((CH,), jnp.int32),
            pltpu.VMEM((NB, 16), jnp.int32),
            pltpu.VMEM((NB, 16), jnp.int32),
            pltpu.VMEM((2, BS, D_MODEL), jnp.float32),
            pltpu.VMEM((2, 32), jnp.int32),
            pltpu.SemaphoreType.DMA,
            pltpu.SemaphoreType.DMA,
        ],
        compiler_params=pltpu.CompilerParams(needs_layout_passes=False),
    )
    return f(eid, pr, x)


def _ffn_body(sref, x_ref, wg_ref, wu_ref, wd_ref, y_ref, acc_ref):
    f = pl.program_id(0)
    t = pl.program_id(1)
    valid = sref[1, t] == 1

    @pl.when(valid)
    def _():
        xb = x_ref[...]
        g = lax.dot_general(xb, wg_ref[0], (((1,), (1,)), ((), ())),
                            preferred_element_type=jnp.float32)
        u = lax.dot_general(xb, wu_ref[0], (((1,), (1,)), ((), ())),
                            preferred_element_type=jnp.float32)
        h = (g * jax.nn.sigmoid(g)) * u
        yp = lax.dot_general(h, wd_ref[0], (((1,), (1,)), ((), ())),
                             preferred_element_type=jnp.float32)

        @pl.when(f == 0)
        def _():
            acc_ref[t] = yp

        @pl.when(jnp.logical_and(f > 0, f < NF - 1))
        def _():
            acc_ref[t] = acc_ref[t] + yp

        @pl.when(f == NF - 1)
        def _():
            y_ref[...] = acc_ref[t] + yp


def _run_ffn(sinfo, X_disp, W_gu, W_d):
    grid = (NF, NT)
    grid_spec = pltpu.PrefetchScalarGridSpec(
        num_scalar_prefetch=1,
        grid=grid,
        in_specs=[
            pl.BlockSpec((TM, D_MODEL), lambda f, t, s: (t, 0)),
            pl.BlockSpec((1, TF, D_MODEL), lambda f, t, s: (s[0, t], f, 0)),
            pl.BlockSpec((1, TF, D_MODEL), lambda f, t, s: (s[0, t], NF + f, 0)),
            pl.BlockSpec((1, D_MODEL, TF), lambda f, t, s: (s[0, t], 0, f)),
        ],
        out_specs=pl.BlockSpec((TM, D_MODEL), lambda f, t, s: (t, 0)),
        scratch_shapes=[pltpu.VMEM((NT, TM, D_MODEL), jnp.float32)],
    )
    return pl.pallas_call(
        _ffn_body,
        grid_spec=grid_spec,
        out_shape=jax.ShapeDtypeStruct((PAD_ROWS, D_MODEL), jnp.float32),
        compiler_params=pltpu.CompilerParams(
            dimension_semantics=("arbitrary", "arbitrary"),
        ),
    )(sinfo, X_disp, W_gu, W_gu, W_d)


def _combine_body(y_hbm, dst_hbm, out_hbm, dst_v, idxr, ybuf, obuf,
                  gsem, ssem):
    wid = lax.axis_index("s") * 2 + lax.axis_index("c")
    base = wid * CH
    tok_base = wid * TOK_CH

    pltpu.sync_copy(dst_hbm.at[pl.ds(base, CH)], dst_v)
    for b in range(NB):
        idxr[b, :] = dst_v[pl.ds(b * 16, 16)]

    gh = [None] * NB
    sh = [None] * NB
    gh[0] = pltpu.async_copy(y_hbm.at[idxr.at[0]], ybuf.at[0], gsem)
    for b in range(NB):
        cur = b % 2
        gh[b].wait()
        if b + 1 < NB:
            if b >= 1:
                sh[b - 1].wait()
            gh[b + 1] = pltpu.async_copy(
                y_hbm.at[idxr.at[b + 1]], ybuf.at[(b + 1) % 2], gsem)

        def add_row(r, carry):
            for j in range(D_MODEL // 16):
                sl = pl.ds(j * 16, 16)
                obuf[cur, r, sl] = ybuf[cur, 2 * r, sl] + ybuf[cur, 2 * r + 1, sl]
            return carry

        lax.fori_loop(0, BS // TOP_K, add_row, 0)
        sh[b] = pltpu.async_copy(
            obuf.at[cur],
            out_hbm.at[pl.ds(tok_base + b * (BS // TOP_K), BS // TOP_K)],
            ssem)
    sh[NB - 2].wait()
    sh[NB - 1].wait()


def _run_combine(Y_disp, dst):
    mesh = plsc.VectorSubcoreMesh(core_axis_name="c", subcore_axis_name="s",
                                   num_cores=2, num_subcores=16)
    f = pl.kernel(
        _combine_body,
        out_type=jax.ShapeDtypeStruct((N_TOKENS, D_MODEL), jnp.float32),
        mesh=mesh,
        scratch_types=[
            pltpu.VMEM((CH,), jnp.int32),
            pltpu.VMEM((NB, 16), jnp.int32),
            pltpu.VMEM((2, BS, D_MODEL), jnp.float32),
            pltpu.VMEM((2, BS // TOP_K, D_MODEL), jnp.float32),
            pltpu.SemaphoreType.DMA,
            pltpu.SemaphoreType.DMA,
        ],
        compiler_params=pltpu.CompilerParams(needs_layout_passes=False),
    )
    return f(Y_disp, dst)


def kernel(x, W_router, W_gu, W_d):
    ei, pp = _run_router(x, W_router)
    eid = ei.reshape(-1)
    pr = pp.reshape(-1)

    dst, X_disp, sinfo_p = _run_dispatch(eid, pr, x)
    sinfo = sinfo_p[:, :NT]

    Y_disp = _run_ffn(sinfo, X_disp, W_gu, W_d)
    return _run_combine(Y_disp, dst)
